# baseline (speedup 1.0000x reference)
import functools
import math

import jax
import jax.numpy as jnp
from jax import lax
from jax.experimental import pallas as pl
from jax.experimental.pallas import tpu as pltpu
from jax.experimental.pallas import tpu_sc as plsc

H = 128
NC, NS, LANES = 2, 16, 16
B = 80
BG = 80
USABLE = 8960
ALLOC = USABLE + 16
TRASH = USABLE
DEV_CHUNK = NC * USABLE
ZROWS = 56
SUB_ROWS = USABLE // NS

N_WORD, N_ENT, N_SENT = 100000, 50000, 20000


def _make_agg(n_src, n_dst, n_edges, interpret=False):
  P = math.ceil(n_dst / DEV_CHUNK)
  n_pad = P * DEV_CHUNK
  nblk = n_edges // B
  blk_per_sub = math.ceil(nblk / NS)
  cap = blk_per_sub * B + B
  mesh = plsc.VectorSubcoreMesh(core_axis_name="c", subcore_axis_name="s",
                                num_cores=NC, num_subcores=NS)

  @functools.partial(
      pl.kernel,
      out_type=jax.ShapeDtypeStruct((n_pad, H), jnp.float32),
      mesh=mesh,
      interpret=interpret,
      compiler_params=pltpu.CompilerParams(needs_layout_passes=False),
      scratch_types=[
          pltpu.VMEM_SHARED((ALLOC, H), jnp.float32),
          pltpu.VMEM((ZROWS, H), jnp.float32),
          pltpu.VMEM((B,), jnp.int32),
          pltpu.VMEM((B,), jnp.int32),
          pltpu.VMEM((cap,), jnp.int32),
          pltpu.VMEM((cap,), jnp.int32),
          pltpu.VMEM((1, B), jnp.int32),
          pltpu.VMEM((B, H), jnp.float32),
          pltpu.SemaphoreType.DMA,
      ],
  )
  def agg(table, src, dst, zeros_h, acc_out,
          acc_sh, zbuf, sstage, dstage, srcbuf, dstbuf, idxrow, rows, sem):
    cid = lax.axis_index("c")
    sid = lax.axis_index("s")
    pltpu.sync_copy(zeros_h, zbuf)
    lane = lax.iota(jnp.int32, LANES)
    trash_v = jnp.full((LANES,), TRASH, jnp.int32)
    nb_me = (nblk - sid + NS - 1) // NS

    for p in range(P):
      lo = p * DEV_CHUNK + cid * USABLE
      for z in range(SUB_ROWS // ZROWS):
        pltpu.sync_copy(zbuf, acc_sh.at[pl.ds(sid * SUB_ROWS + z * ZROWS, ZROWS)])
      plsc.subcore_barrier()

      def compact(i, cnt):
        base = (sid + i * NS) * B
        pltpu.sync_copy(src.at[pl.ds(base, B)], sstage)
        pltpu.sync_copy(dst.at[pl.ds(base, B)], dstage)
        for g in range(B // LANES):
          sv = sstage[pl.ds(g * LANES, LANES)]
          dloc = dstage[pl.ds(g * LANES, LANES)] - lo
          m = (dloc >= 0) & (dloc < USABLE)
          mi = jnp.where(m, 1, 0).astype(jnp.int32)
          pos = cnt + plsc.cumsum(mi) - 1
          plsc.store_scatter(srcbuf, [pos], sv, mask=m)
          plsc.store_scatter(dstbuf, [pos], dloc, mask=m)
          cnt = cnt + jnp.sum(mi)
        return cnt

      cnt = lax.fori_loop(0, nb_me, compact, jnp.int32(0))

      for g in range(B // LANES):
        srcbuf[pl.ds(cnt + g * LANES, LANES)] = (lane + g * LANES) & 63
        dstbuf[pl.ds(cnt + g * LANES, LANES)] = trash_v

      def gath(j, _):
        off = j * B
        pltpu.async_copy(table.at[srcbuf.at[pl.ds(off, B)]], rows, sem).wait()
        for g in range(B // LANES):
          idxrow.at[0][pl.ds(g * LANES, LANES)] = (
              dstbuf[pl.ds(off + g * LANES, LANES)])
        pltpu.sync_copy(rows, acc_sh.at[idxrow.at[0]], add=True)
        return 0

      lax.fori_loop(0, (cnt + B - 1) // B, gath, 0)
      plsc.subcore_barrier()
      out_base = p * DEV_CHUNK + cid * USABLE + sid * SUB_ROWS
      pltpu.sync_copy(acc_sh.at[pl.ds(sid * SUB_ROWS, SUB_ROWS)],
                      acc_out.at[pl.ds(out_base, SUB_ROWS)])
      plsc.subcore_barrier()

  def run(table, src, dst):
    zeros_h = jnp.zeros((ZROWS, H), jnp.float32)
    return agg(table, src, dst, zeros_h)

  return run


def _make_pair_gather(n_a, n_b, n_edges, interpret=False):
  nblk = n_edges // BG
  mesh = plsc.VectorSubcoreMesh(core_axis_name="c", subcore_axis_name="s",
                                num_cores=NC, num_subcores=NS)
  nw = NC * NS

  @functools.partial(
      pl.kernel,
      out_type=(jax.ShapeDtypeStruct((n_edges, H), jnp.float32),
                jax.ShapeDtypeStruct((n_edges, H), jnp.float32)),
      mesh=mesh,
      interpret=interpret,
      compiler_params=pltpu.CompilerParams(needs_layout_passes=False),
      scratch_types=[
          pltpu.VMEM((BG,), jnp.int32),
          pltpu.VMEM((BG, H), jnp.float32),
          pltpu.SemaphoreType.DMA,
      ],
  )
  def gk(tab_a, tab_b, ia, ib, out_a, out_b, stage, rows, sem):
    wid = lax.axis_index("s") * NC + lax.axis_index("c")
    nb_me = (nblk - wid + nw - 1) // nw

    def body(i, _):
      base = (wid + i * nw) * BG
      pltpu.sync_copy(ia.at[pl.ds(base, BG)], stage)
      pltpu.async_copy(tab_a.at[stage], rows, sem).wait()
      pltpu.sync_copy(rows, out_a.at[pl.ds(base, BG)])
      pltpu.sync_copy(ib.at[pl.ds(base, BG)], stage)
      pltpu.async_copy(tab_b.at[stage], rows, sem).wait()
      pltpu.sync_copy(rows, out_b.at[pl.ds(base, BG)])
      return 0

    lax.fori_loop(0, nb_me, body, 0)

  return gk


BM = 1000


def _mm_kernel(x_ref, w_ref, o_ref):
  o_ref[...] = jnp.dot(x_ref[...], w_ref[...],
                       preferred_element_type=jnp.float32)


def _matmul(x, wt):
  n, k = x.shape
  return pl.pallas_call(
      _mm_kernel,
      grid=(n // BM,),
      in_specs=[pl.BlockSpec((BM, k), lambda i: (i, 0)),
                pl.BlockSpec((k, H), lambda i: (0, 0))],
      out_specs=pl.BlockSpec((BM, H), lambda i: (i, 0)),
      out_shape=jax.ShapeDtypeStruct((n, H), jnp.float32),
  )(x, wt)


def _proj_kernel(x_ref, w_ref, b_ref, e_ref, o_ref):
  o_ref[...] = (jnp.dot(x_ref[...], w_ref[...],
                        preferred_element_type=jnp.float32)
                + b_ref[...] + e_ref[...])


def _proj(x, wt, b, emb):
  n, k = x.shape
  return pl.pallas_call(
      _proj_kernel,
      grid=(n // BM,),
      in_specs=[pl.BlockSpec((BM, k), lambda i: (i, 0)),
                pl.BlockSpec((k, H), lambda i: (0, 0)),
                pl.BlockSpec((1, H), lambda i: (0, 0)),
                pl.BlockSpec((BM, H), lambda i: (i, 0))],
      out_specs=pl.BlockSpec((BM, H), lambda i: (i, 0)),
      out_shape=jax.ShapeDtypeStruct((n, H), jnp.float32),
  )(x, wt, b, emb)


def _combine_word_kernel(relu, a1, c1, a2, c2, xp, wt, b, o_ref):
  r1 = a1[...] / jnp.maximum(c1[...], 1.0)
  r2 = a2[...] / jnp.maximum(c2[...], 1.0)
  out = (r1 + r2 + jnp.dot(xp[...], wt[...],
                           preferred_element_type=jnp.float32) + b[...])
  o_ref[...] = jnp.maximum(out, 0.0) if relu else out


def _combine_word(acc1, cnt1, acc2, cnt2, xprev, wt, b, relu):
  n = xprev.shape[0]
  return pl.pallas_call(
      functools.partial(_combine_word_kernel, relu),
      grid=(n // BM,),
      in_specs=[pl.BlockSpec((BM, H), lambda i: (i, 0)),
                pl.BlockSpec((BM, 1), lambda i: (i, 0)),
                pl.BlockSpec((BM, H), lambda i: (i, 0)),
                pl.BlockSpec((BM, 1), lambda i: (i, 0)),
                pl.BlockSpec((BM, H), lambda i: (i, 0)),
                pl.BlockSpec((H, H), lambda i: (0, 0)),
                pl.BlockSpec((1, H), lambda i: (0, 0))],
      out_specs=pl.BlockSpec((BM, H), lambda i: (i, 0)),
      out_shape=jax.ShapeDtypeStruct((n, H), jnp.float32),
  )(acc1, cnt1, acc2, cnt2, xprev, wt, b)


def _combine_small_kernel(relu, a, c, xp, wlt, wrt, b, o_ref):
  mean = a[...] / jnp.maximum(c[...], 1.0)
  out = (jnp.dot(mean, wlt[...], preferred_element_type=jnp.float32)
         + jnp.dot(xp[...], wrt[...], preferred_element_type=jnp.float32)
         + b[...])
  o_ref[...] = jnp.maximum(out, 0.0) if relu else out


def _combine_small(acc, cnt, xprev, wlt, wrt, b, relu):
  n = xprev.shape[0]
  return pl.pallas_call(
      functools.partial(_combine_small_kernel, relu),
      grid=(n // BM,),
      in_specs=[pl.BlockSpec((BM, H), lambda i: (i, 0)),
                pl.BlockSpec((BM, 1), lambda i: (i, 0)),
                pl.BlockSpec((BM, H), lambda i: (i, 0)),
                pl.BlockSpec((H, H), lambda i: (0, 0)),
                pl.BlockSpec((H, H), lambda i: (0, 0)),
                pl.BlockSpec((1, H), lambda i: (0, 0))],
      out_specs=pl.BlockSpec((BM, H), lambda i: (i, 0)),
      out_shape=jax.ShapeDtypeStruct((n, H), jnp.float32),
  )(acc, cnt, xprev, wlt, wrt, b)


def _dot_kernel(a_ref, b_ref, o_ref):
  o_ref[...] = jnp.sum(a_ref[...] * b_ref[...], axis=1, keepdims=True)


def _rowdot(a, b):
  n = a.shape[0]
  out = pl.pallas_call(
      _dot_kernel,
      grid=(n // BM,),
      in_specs=[pl.BlockSpec((BM, H), lambda i: (i, 0)),
                pl.BlockSpec((BM, H), lambda i: (i, 0))],
      out_specs=pl.BlockSpec((BM, 1), lambda i: (i, 0)),
      out_shape=jax.ShapeDtypeStruct((n, 1), jnp.float32),
  )(a, b)
  return out.reshape(-1)


_agg_ew = _make_agg(N_ENT, N_WORD, 200000)
_agg_sw = _make_agg(N_SENT, N_WORD, 100000)
_agg_we = _make_agg(N_WORD, N_ENT, 200000)
_agg_ws = _make_agg(N_WORD, N_SENT, 100000)
_pair_gather = _make_pair_gather(N_WORD, N_ENT, 100000)


def kernel(word_x, word_node_id, entity_node_id, sentence_node_id,
           ei_word_entity, ei_entity_word, ei_word_sentence, ei_sentence_word,
           edge_label_index, params):
  p = params
  xe = p['entity_emb']
  xs = p['entity_emb'][:N_SENT]
  wx = jnp.pad(word_x, ((0, 0), (0, 9)))
  wt = jnp.pad(p['word_lin_W'].T, ((0, 9), (0, 0)))
  xw = _proj(wx, wt, p['word_lin_b'][None, :], p['word_emb'])

  we_s, we_d = ei_word_entity[0], ei_word_entity[1]
  ew_s, ew_d = ei_entity_word[0], ei_entity_word[1]
  ws_s, ws_d = ei_word_sentence[0], ei_word_sentence[1]
  sw_s, sw_d = ei_sentence_word[0], ei_sentence_word[1]

  ones_tab = jnp.ones((N_WORD, H), jnp.float32)
  cnt_ew = _agg_ew(ones_tab[:N_ENT], ew_s, ew_d)[:N_WORD, 0:1]
  cnt_sw = _agg_sw(ones_tab[:N_SENT], sw_s, sw_d)[:N_WORD, 0:1]
  cnt_we = _agg_we(ones_tab, we_s, we_d)[:N_ENT, 0:1]
  cnt_ws = _agg_ws(ones_tab, ws_s, ws_d)[:N_SENT, 0:1]

  hw_prev, he_prev, hs_prev = xw, xe, xs
  for lyr, relu in (('l1', True), ('l2', False)):
    def g(rel, s):
      return p[lyr + '_' + rel + '_' + s]

    y_ew = _matmul(he_prev, g('ew', 'Wl').T)
    y_sw = _matmul(hs_prev, g('sw', 'Wl').T)
    acc_ew = _agg_ew(y_ew, ew_s, ew_d)
    acc_sw = _agg_sw(y_sw, sw_s, sw_d)
    acc_we = _agg_we(hw_prev, we_s, we_d)

    wr_sum = (g('ew', 'Wr') + g('sw', 'Wr')).T
    b_sum = (g('ew', 'bl') + g('sw', 'bl'))[None, :]
    hw = _combine_word(acc_ew[:N_WORD], cnt_ew,
                       acc_sw[:N_WORD], cnt_sw,
                       hw_prev, wr_sum, b_sum, relu)
    he = _combine_small(acc_we[:N_ENT], cnt_we, he_prev,
                        g('we', 'Wl').T, g('we', 'Wr').T,
                        g('we', 'bl')[None, :], relu)
    if relu:
      acc_ws = _agg_ws(hw_prev, ws_s, ws_d)
      hs = _combine_small(acc_ws[:N_SENT], cnt_ws, hs_prev,
                          g('ws', 'Wl').T, g('ws', 'Wr').T,
                          g('ws', 'bl')[None, :], relu)
      hs_prev = hs
    hw_prev, he_prev = hw, he

  ga, gb = _pair_gather(hw_prev, he_prev,
                        edge_label_index[0], edge_label_index[1])
  return _rowdot(ga, gb)


# speedup vs baseline: 2.8516x; 2.8516x over previous
import functools
import math

import jax
import jax.numpy as jnp
from jax import lax
from jax.experimental import pallas as pl
from jax.experimental.pallas import tpu as pltpu
from jax.experimental.pallas import tpu_sc as plsc

H = 128
NC, NS, LANES = 2, 16, 16
B = 80
BG = 80
USABLE = 6784
ALLOC = USABLE + 16
TRASH = USABLE
DEV_CHUNK = NC * USABLE
ZROWS = 16
SUB_ROWS = USABLE // NS

N_WORD, N_ENT, N_SENT = 100000, 50000, 20000


def _make_agg(n_src, n_dst, n_edges_pad, interpret=False):
  P = math.ceil(n_dst / DEV_CHUNK)
  n_pad = P * DEV_CHUNK
  esub = n_edges_pad // NS
  cap = esub + 2 * B
  mesh = plsc.VectorSubcoreMesh(core_axis_name="c", subcore_axis_name="s",
                                num_cores=NC, num_subcores=NS)

  @functools.partial(
      pl.kernel,
      out_type=jax.ShapeDtypeStruct((n_pad, H), jnp.float32),
      mesh=mesh,
      interpret=interpret,
      compiler_params=pltpu.CompilerParams(needs_layout_passes=False),
      scratch_types=[
          pltpu.VMEM_SHARED((ALLOC, H), jnp.float32),
          pltpu.VMEM((ZROWS, H), jnp.float32),
          pltpu.VMEM((esub,), jnp.int32),
          pltpu.VMEM((esub,), jnp.int32),
          pltpu.VMEM((cap,), jnp.int32),
          pltpu.VMEM((cap,), jnp.int32),
          pltpu.VMEM((1, B), jnp.int32),
          pltpu.VMEM((1, B), jnp.int32),
          pltpu.VMEM((B, H), jnp.float32),
          pltpu.VMEM((B, H), jnp.float32),
          pltpu.SemaphoreType.DMA,
          pltpu.SemaphoreType.DMA,
      ],
  )
  def agg(table, src, dst, zeros_h, acc_out,
          acc_sh, zbuf, sall, dall, srcbuf, dstbuf, idx0, idx1,
          rows0, rows1, semA, semB):
    cid = lax.axis_index("c")
    sid = lax.axis_index("s")
    pltpu.sync_copy(zeros_h, zbuf)
    pltpu.sync_copy(src.at[pl.ds(sid * esub, esub)], sall)
    pltpu.sync_copy(dst.at[pl.ds(sid * esub, esub)], dall)
    lane = lax.iota(jnp.int32, LANES)
    trash_v = jnp.full((LANES,), TRASH, jnp.int32)

    for p in range(P):
      lo = p * DEV_CHUNK + cid * USABLE
      for z in range(SUB_ROWS // ZROWS):
        pltpu.sync_copy(zbuf, acc_sh.at[pl.ds(sid * SUB_ROWS + z * ZROWS, ZROWS)])
      plsc.subcore_barrier()

      def compact(i, cnt):
        sv = sall[pl.ds(i * LANES, LANES)]
        dloc = dall[pl.ds(i * LANES, LANES)] - lo
        m = (dloc >= 0) & (dloc < USABLE)
        mi = jnp.where(m, 1, 0).astype(jnp.int32)
        pos = cnt + plsc.cumsum(mi) - 1
        plsc.store_scatter(srcbuf, [pos], sv, mask=m)
        plsc.store_scatter(dstbuf, [pos], dloc, mask=m)
        return cnt + jnp.sum(mi)

      cnt = lax.fori_loop(0, esub // LANES, compact, jnp.int32(0))

      for g in range(2 * B // LANES):
        srcbuf[pl.ds(cnt + g * LANES, LANES)] = (lane + g * LANES) & 63
        dstbuf[pl.ds(cnt + g * LANES, LANES)] = trash_v

      def gath(j, _):
        o0 = j * 2 * B
        o1 = o0 + B
        for g in range(B // LANES):
          idx0.at[0][pl.ds(g * LANES, LANES)] = dstbuf[pl.ds(o0 + g * LANES, LANES)]
          idx1.at[0][pl.ds(g * LANES, LANES)] = dstbuf[pl.ds(o1 + g * LANES, LANES)]
        d0 = pltpu.async_copy(table.at[srcbuf.at[pl.ds(o0, B)]], rows0, semA)
        d1 = pltpu.async_copy(table.at[srcbuf.at[pl.ds(o1, B)]], rows1, semB)
        d0.wait()
        pltpu.sync_copy(rows0, acc_sh.at[idx0.at[0]], add=True)
        d1.wait()
        pltpu.sync_copy(rows1, acc_sh.at[idx1.at[0]], add=True)
        return 0

      lax.fori_loop(0, (cnt + 2 * B - 1) // (2 * B), gath, 0)
      plsc.subcore_barrier()
      out_base = p * DEV_CHUNK + cid * USABLE + sid * SUB_ROWS
      pltpu.sync_copy(acc_sh.at[pl.ds(sid * SUB_ROWS, SUB_ROWS)],
                      acc_out.at[pl.ds(out_base, SUB_ROWS)])
      plsc.subcore_barrier()

  def run(table, src, dst):
    zeros_h = jnp.zeros((ZROWS, H), jnp.float32)
    return agg(table, src, dst, zeros_h)

  return run


def _make_pair_gather(n_a, n_b, n_edges_pad, interpret=False):
  nw = NC * NS
  esub = n_edges_pad // nw
  mesh = plsc.VectorSubcoreMesh(core_axis_name="c", subcore_axis_name="s",
                                num_cores=NC, num_subcores=NS)

  @functools.partial(
      pl.kernel,
      out_type=(jax.ShapeDtypeStruct((n_edges_pad, H), jnp.float32),
                jax.ShapeDtypeStruct((n_edges_pad, H), jnp.float32)),
      mesh=mesh,
      interpret=interpret,
      compiler_params=pltpu.CompilerParams(needs_layout_passes=False),
      scratch_types=[
          pltpu.VMEM((n_edges_pad // (NC * NS),), jnp.int32),
          pltpu.VMEM((n_edges_pad // (NC * NS),), jnp.int32),
          pltpu.VMEM((B, H), jnp.float32),
          pltpu.VMEM((B, H), jnp.float32),
          pltpu.SemaphoreType.DMA,
          pltpu.SemaphoreType.DMA,
      ],
  )
  def gk(tab_a, tab_b, ia, ib, out_a, out_b, sa, sb, rows0, rows1,
         semA, semB):
    wid = lax.axis_index("s") * NC + lax.axis_index("c")
    base = wid * esub
    pltpu.sync_copy(ia.at[pl.ds(base, esub)], sa)
    pltpu.sync_copy(ib.at[pl.ds(base, esub)], sb)

    def body(j, _):
      o = j * B
      d0 = pltpu.async_copy(tab_a.at[sa.at[pl.ds(o, B)]], rows0, semA)
      d1 = pltpu.async_copy(tab_b.at[sb.at[pl.ds(o, B)]], rows1, semB)
      d0.wait()
      pltpu.sync_copy(rows0, out_a.at[pl.ds(base + o, B)])
      d1.wait()
      pltpu.sync_copy(rows1, out_b.at[pl.ds(base + o, B)])
      return 0

    lax.fori_loop(0, esub // B, body, 0)

  return gk


BM = 1000


def _mm_kernel(x_ref, w_ref, o_ref):
  o_ref[...] = jnp.dot(x_ref[...], w_ref[...],
                       preferred_element_type=jnp.float32)


def _matmul(x, wt):
  n, k = x.shape
  return pl.pallas_call(
      _mm_kernel,
      grid=(n // BM,),
      in_specs=[pl.BlockSpec((BM, k), lambda i: (i, 0)),
                pl.BlockSpec((k, H), lambda i: (0, 0))],
      out_specs=pl.BlockSpec((BM, H), lambda i: (i, 0)),
      out_shape=jax.ShapeDtypeStruct((n, H), jnp.float32),
  )(x, wt)


def _proj_kernel(x_ref, w_ref, b_ref, e_ref, o_ref):
  o_ref[...] = (jnp.dot(x_ref[...], w_ref[...],
                        preferred_element_type=jnp.float32)
                + b_ref[...] + e_ref[...])


def _proj(x, wt, b, emb):
  n, k = x.shape
  return pl.pallas_call(
      _proj_kernel,
      grid=(n // BM,),
      in_specs=[pl.BlockSpec((BM, k), lambda i: (i, 0)),
                pl.BlockSpec((k, H), lambda i: (0, 0)),
                pl.BlockSpec((1, H), lambda i: (0, 0)),
                pl.BlockSpec((BM, H), lambda i: (i, 0))],
      out_specs=pl.BlockSpec((BM, H), lambda i: (i, 0)),
      out_shape=jax.ShapeDtypeStruct((n, H), jnp.float32),
  )(x, wt, b, emb)


def _combine_word_kernel(relu, a1, c1, a2, c2, xp, wt, b, o_ref):
  r1 = a1[...] / jnp.maximum(c1[...], 1.0)
  r2 = a2[...] / jnp.maximum(c2[...], 1.0)
  out = (r1 + r2 + jnp.dot(xp[...], wt[...],
                           preferred_element_type=jnp.float32) + b[...])
  o_ref[...] = jnp.maximum(out, 0.0) if relu else out


def _combine_word(acc1, cnt1, acc2, cnt2, xprev, wt, b, relu):
  n = xprev.shape[0]
  return pl.pallas_call(
      functools.partial(_combine_word_kernel, relu),
      grid=(n // BM,),
      in_specs=[pl.BlockSpec((BM, H), lambda i: (i, 0)),
                pl.BlockSpec((BM, 1), lambda i: (i, 0)),
                pl.BlockSpec((BM, H), lambda i: (i, 0)),
                pl.BlockSpec((BM, 1), lambda i: (i, 0)),
                pl.BlockSpec((BM, H), lambda i: (i, 0)),
                pl.BlockSpec((H, H), lambda i: (0, 0)),
                pl.BlockSpec((1, H), lambda i: (0, 0))],
      out_specs=pl.BlockSpec((BM, H), lambda i: (i, 0)),
      out_shape=jax.ShapeDtypeStruct((n, H), jnp.float32),
  )(acc1, cnt1, acc2, cnt2, xprev, wt, b)


def _combine_small_kernel(relu, a, c, xp, wlt, wrt, b, o_ref):
  mean = a[...] / jnp.maximum(c[...], 1.0)
  out = (jnp.dot(mean, wlt[...], preferred_element_type=jnp.float32)
         + jnp.dot(xp[...], wrt[...], preferred_element_type=jnp.float32)
         + b[...])
  o_ref[...] = jnp.maximum(out, 0.0) if relu else out


def _combine_small(acc, cnt, xprev, wlt, wrt, b, relu):
  n = xprev.shape[0]
  return pl.pallas_call(
      functools.partial(_combine_small_kernel, relu),
      grid=(n // BM,),
      in_specs=[pl.BlockSpec((BM, H), lambda i: (i, 0)),
                pl.BlockSpec((BM, 1), lambda i: (i, 0)),
                pl.BlockSpec((BM, H), lambda i: (i, 0)),
                pl.BlockSpec((H, H), lambda i: (0, 0)),
                pl.BlockSpec((H, H), lambda i: (0, 0)),
                pl.BlockSpec((1, H), lambda i: (0, 0))],
      out_specs=pl.BlockSpec((BM, H), lambda i: (i, 0)),
      out_shape=jax.ShapeDtypeStruct((n, H), jnp.float32),
  )(acc, cnt, xprev, wlt, wrt, b)


def _dot_kernel(a_ref, b_ref, o_ref):
  o_ref[...] = jnp.sum(a_ref[...] * b_ref[...], axis=1, keepdims=True)


def _rowdot(a, b):
  n = a.shape[0]
  out = pl.pallas_call(
      _dot_kernel,
      grid=(n // BM,),
      in_specs=[pl.BlockSpec((BM, H), lambda i: (i, 0)),
                pl.BlockSpec((BM, H), lambda i: (i, 0))],
      out_specs=pl.BlockSpec((BM, 1), lambda i: (i, 0)),
      out_shape=jax.ShapeDtypeStruct((n, 1), jnp.float32),
  )(a, b)
  return out.reshape(-1)


EPAD2 = 200320
EPAD1 = 100096
LPAD = 102400

_agg_ew = _make_agg(N_ENT, N_WORD, EPAD2)
_agg_sw = _make_agg(N_SENT, N_WORD, EPAD1)
_agg_we = _make_agg(N_WORD, N_ENT, EPAD2)
_agg_ws = _make_agg(N_WORD, N_SENT, EPAD1)
_pair_gather = _make_pair_gather(N_WORD, N_ENT, LPAD)


def kernel(word_x, word_node_id, entity_node_id, sentence_node_id,
           ei_word_entity, ei_entity_word, ei_word_sentence, ei_sentence_word,
           edge_label_index, params):
  p = params
  xe = p['entity_emb']
  xs = p['entity_emb'][:N_SENT]
  wx = jnp.pad(word_x, ((0, 0), (0, 9)))
  wt = jnp.pad(p['word_lin_W'].T, ((0, 9), (0, 0)))
  xw = _proj(wx, wt, p['word_lin_b'][None, :], p['word_emb'])

  def pad_edges(ei, n_pad_e):
    e = ei.shape[1]
    pad = n_pad_e - e
    spread = (jnp.arange(pad, dtype=jnp.int32) & 63)
    sentinel = jnp.full((pad,), 10 * DEV_CHUNK, jnp.int32)
    return (jnp.concatenate([ei[0], spread]),
            jnp.concatenate([ei[1], sentinel]))

  we_s, we_d = pad_edges(ei_word_entity, EPAD2)
  ew_s, ew_d = pad_edges(ei_entity_word, EPAD2)
  ws_s, ws_d = pad_edges(ei_word_sentence, EPAD1)
  sw_s, sw_d = pad_edges(ei_sentence_word, EPAD1)
  la = jnp.concatenate([edge_label_index[0],
                        (jnp.arange(LPAD - 100000, dtype=jnp.int32) & 63)])
  lb = jnp.concatenate([edge_label_index[1],
                        (jnp.arange(LPAD - 100000, dtype=jnp.int32) & 63)])

  def chained(fn, *args):
    nonlocal tok
    first = lax.optimization_barrier((args[0], tok))[0]
    out = fn(first, *args[1:])
    tok = out[0, 0]
    return out

  tok = jnp.float32(0.0)

  ones_tab = jnp.ones((N_WORD, H), jnp.float32) * (1.0 + 0.0 * word_x[0, 0])
  def _cnt_jax(dstx, n_dst):
    return jax.ops.segment_sum(jnp.ones((dstx.shape[0],), jnp.float32), dstx,
                               num_segments=n_dst)[:, None]

  cnt_ew = chained(_agg_ew, ones_tab[:N_ENT], ew_s, ew_d)[:N_WORD, 0:1]
  cnt_sw = chained(_agg_sw, ones_tab[:N_SENT], sw_s, sw_d)[:N_WORD, 0:1]
  cnt_we = chained(_agg_we, ones_tab, we_s, we_d)[:N_ENT, 0:1]
  cnt_ws = _cnt_jax(ei_word_sentence[1], N_SENT)
  cnt_delta = jnp.float32(0.0)

  hw_prev, he_prev, hs_prev = xw, xe, xs
  for lyr, relu in (('l1', True), ('l2', False)):
    def g(rel, s):
      return p[lyr + '_' + rel + '_' + s]

    y_ew = _matmul(he_prev, g('ew', 'Wl').T)
    y_sw = _matmul(hs_prev, g('sw', 'Wl').T)
    acc_ew = chained(_agg_ew, y_ew, ew_s, ew_d)
    acc_sw = chained(_agg_sw, y_sw, sw_s, sw_d)
    acc_we = chained(_agg_we, hw_prev, we_s, we_d)

    wr_sum = (g('ew', 'Wr') + g('sw', 'Wr')).T
    b_sum = (g('ew', 'bl') + g('sw', 'bl'))[None, :]
    hw = _combine_word(acc_ew[:N_WORD], cnt_ew,
                       acc_sw[:N_WORD], cnt_sw,
                       hw_prev, wr_sum, b_sum, relu)
    he = _combine_small(acc_we[:N_ENT], cnt_we, he_prev,
                        g('we', 'Wl').T, g('we', 'Wr').T,
                        g('we', 'bl')[None, :], relu)
    if relu:
      acc_ws = chained(_agg_ws, hw_prev, ws_s, ws_d)
      hs = _combine_small(acc_ws[:N_SENT], cnt_ws, hs_prev,
                          g('ws', 'Wl').T, g('ws', 'Wr').T,
                          g('ws', 'bl')[None, :], relu)
      hs_prev = hs
    hw_prev, he_prev = hw, he

  hw_prev = lax.optimization_barrier((hw_prev, tok))[0]
  ga, gb = _pair_gather(hw_prev, he_prev, la, lb)
  return _rowdot(ga[:100000], gb[:100000]) + cnt_delta


# speedup vs baseline: 2.8519x; 1.0001x over previous
import functools
import math

import jax
import jax.numpy as jnp
from jax import lax
from jax.experimental import pallas as pl
from jax.experimental.pallas import tpu as pltpu
from jax.experimental.pallas import tpu_sc as plsc

H = 128
NC, NS, LANES = 2, 16, 16
B = 80
BG = 80
USABLE = 6784
ALLOC = USABLE + 16
TRASH = USABLE
DEV_CHUNK = NC * USABLE
ZROWS = 16
SUB_ROWS = USABLE // NS

N_WORD, N_ENT, N_SENT = 100000, 50000, 20000


def _make_agg(n_src, n_dst, n_edges_pad, interpret=False):
  P = math.ceil(n_dst / DEV_CHUNK)
  n_pad = P * DEV_CHUNK
  esub = n_edges_pad // NS
  cap = esub + 2 * B
  mesh = plsc.VectorSubcoreMesh(core_axis_name="c", subcore_axis_name="s",
                                num_cores=NC, num_subcores=NS)

  @functools.partial(
      pl.kernel,
      out_type=jax.ShapeDtypeStruct((n_pad, H), jnp.float32),
      mesh=mesh,
      interpret=interpret,
      compiler_params=pltpu.CompilerParams(needs_layout_passes=False),
      scratch_types=[
          pltpu.VMEM_SHARED((ALLOC, H), jnp.float32),
          pltpu.VMEM((ZROWS, H), jnp.float32),
          pltpu.VMEM((esub,), jnp.int32),
          pltpu.VMEM((esub,), jnp.int32),
          pltpu.VMEM((cap,), jnp.int32),
          pltpu.VMEM((cap,), jnp.int32),
          pltpu.VMEM((1, B), jnp.int32),
          pltpu.VMEM((1, B), jnp.int32),
          pltpu.VMEM((B, H), jnp.float32),
          pltpu.VMEM((B, H), jnp.float32),
          pltpu.SemaphoreType.DMA,
          pltpu.SemaphoreType.DMA,
      ],
  )
  def agg(table, src, dst, zeros_h, acc_out,
          acc_sh, zbuf, sall, dall, srcbuf, dstbuf, idx0, idx1,
          rows0, rows1, semA, semB):
    cid = lax.axis_index("c")
    sid = lax.axis_index("s")
    pltpu.sync_copy(zeros_h, zbuf)
    pltpu.sync_copy(src.at[pl.ds(sid * esub, esub)], sall)
    pltpu.sync_copy(dst.at[pl.ds(sid * esub, esub)], dall)
    lane = lax.iota(jnp.int32, LANES)
    trash_v = jnp.full((LANES,), TRASH, jnp.int32)

    for p in range(P):
      lo = p * DEV_CHUNK + cid * USABLE
      for z in range(SUB_ROWS // ZROWS):
        pltpu.sync_copy(zbuf, acc_sh.at[pl.ds(sid * SUB_ROWS + z * ZROWS, ZROWS)])
      plsc.subcore_barrier()

      def compact(i, cnt):
        sv = sall[pl.ds(i * LANES, LANES)]
        dloc = dall[pl.ds(i * LANES, LANES)] - lo
        m = (dloc >= 0) & (dloc < USABLE)
        mi = jnp.where(m, 1, 0).astype(jnp.int32)
        pos = cnt + plsc.cumsum(mi) - 1
        plsc.store_scatter(srcbuf, [pos], sv, mask=m)
        plsc.store_scatter(dstbuf, [pos], dloc, mask=m)
        return cnt + jnp.sum(mi)

      cnt = lax.fori_loop(0, esub // LANES, compact, jnp.int32(0))

      for g in range(2 * B // LANES):
        srcbuf[pl.ds(cnt + g * LANES, LANES)] = (lane + g * LANES) & 63
        dstbuf[pl.ds(cnt + g * LANES, LANES)] = trash_v

      def gath(j, _):
        o0 = j * 2 * B
        o1 = o0 + B
        for g in range(B // LANES):
          idx0.at[0][pl.ds(g * LANES, LANES)] = dstbuf[pl.ds(o0 + g * LANES, LANES)]
          idx1.at[0][pl.ds(g * LANES, LANES)] = dstbuf[pl.ds(o1 + g * LANES, LANES)]
        d0 = pltpu.async_copy(table.at[srcbuf.at[pl.ds(o0, B)]], rows0, semA)
        d1 = pltpu.async_copy(table.at[srcbuf.at[pl.ds(o1, B)]], rows1, semB)
        d0.wait()
        pltpu.sync_copy(rows0, acc_sh.at[idx0.at[0]], add=True)
        d1.wait()
        pltpu.sync_copy(rows1, acc_sh.at[idx1.at[0]], add=True)
        return 0

      lax.fori_loop(0, (cnt + 2 * B - 1) // (2 * B), gath, 0)
      plsc.subcore_barrier()
      out_base = p * DEV_CHUNK + cid * USABLE + sid * SUB_ROWS
      pltpu.sync_copy(acc_sh.at[pl.ds(sid * SUB_ROWS, SUB_ROWS)],
                      acc_out.at[pl.ds(out_base, SUB_ROWS)])
      plsc.subcore_barrier()

  def run(table, src, dst):
    zeros_h = jnp.zeros((ZROWS, H), jnp.float32)
    return agg(table, src, dst, zeros_h)

  return run


def _make_pair_gather(n_a, n_b, n_edges_pad, interpret=False):
  nw = NC * NS
  esub = n_edges_pad // nw
  mesh = plsc.VectorSubcoreMesh(core_axis_name="c", subcore_axis_name="s",
                                num_cores=NC, num_subcores=NS)

  @functools.partial(
      pl.kernel,
      out_type=(jax.ShapeDtypeStruct((n_edges_pad, H), jnp.float32),
                jax.ShapeDtypeStruct((n_edges_pad, H), jnp.float32)),
      mesh=mesh,
      interpret=interpret,
      compiler_params=pltpu.CompilerParams(needs_layout_passes=False),
      scratch_types=[
          pltpu.VMEM((n_edges_pad // (NC * NS),), jnp.int32),
          pltpu.VMEM((n_edges_pad // (NC * NS),), jnp.int32),
          pltpu.VMEM((B, H), jnp.float32),
          pltpu.VMEM((B, H), jnp.float32),
          pltpu.SemaphoreType.DMA,
          pltpu.SemaphoreType.DMA,
      ],
  )
  def gk(tab_a, tab_b, ia, ib, out_a, out_b, sa, sb, rows0, rows1,
         semA, semB):
    wid = lax.axis_index("s") * NC + lax.axis_index("c")
    base = wid * esub
    pltpu.sync_copy(ia.at[pl.ds(base, esub)], sa)
    pltpu.sync_copy(ib.at[pl.ds(base, esub)], sb)

    def body(j, _):
      o = j * B
      d0 = pltpu.async_copy(tab_a.at[sa.at[pl.ds(o, B)]], rows0, semA)
      d1 = pltpu.async_copy(tab_b.at[sb.at[pl.ds(o, B)]], rows1, semB)
      d0.wait()
      pltpu.sync_copy(rows0, out_a.at[pl.ds(base + o, B)])
      d1.wait()
      pltpu.sync_copy(rows1, out_b.at[pl.ds(base + o, B)])
      return 0

    lax.fori_loop(0, esub // B, body, 0)

  return gk


BM = 1000


def _mm_kernel(x_ref, w_ref, o_ref):
  o_ref[...] = jnp.dot(x_ref[...], w_ref[...],
                       preferred_element_type=jnp.float32)


def _matmul(x, wt):
  n, k = x.shape
  return pl.pallas_call(
      _mm_kernel,
      grid=(n // BM,),
      in_specs=[pl.BlockSpec((BM, k), lambda i: (i, 0)),
                pl.BlockSpec((k, H), lambda i: (0, 0))],
      out_specs=pl.BlockSpec((BM, H), lambda i: (i, 0)),
      out_shape=jax.ShapeDtypeStruct((n, H), jnp.float32),
  )(x, wt)


def _proj_kernel(x_ref, w_ref, b_ref, e_ref, o_ref):
  o_ref[...] = (jnp.dot(x_ref[...], w_ref[...],
                        preferred_element_type=jnp.float32)
                + b_ref[...] + e_ref[...])


def _proj(x, wt, b, emb):
  n, k = x.shape
  return pl.pallas_call(
      _proj_kernel,
      grid=(n // BM,),
      in_specs=[pl.BlockSpec((BM, k), lambda i: (i, 0)),
                pl.BlockSpec((k, H), lambda i: (0, 0)),
                pl.BlockSpec((1, H), lambda i: (0, 0)),
                pl.BlockSpec((BM, H), lambda i: (i, 0))],
      out_specs=pl.BlockSpec((BM, H), lambda i: (i, 0)),
      out_shape=jax.ShapeDtypeStruct((n, H), jnp.float32),
  )(x, wt, b, emb)


def _combine_word_kernel(relu, a1, c1, a2, c2, xp, wt, b, o_ref):
  r1 = a1[...] / jnp.maximum(c1[...], 1.0)
  r2 = a2[...] / jnp.maximum(c2[...], 1.0)
  out = (r1 + r2 + jnp.dot(xp[...], wt[...],
                           preferred_element_type=jnp.float32) + b[...])
  o_ref[...] = jnp.maximum(out, 0.0) if relu else out


def _combine_word(acc1, cnt1, acc2, cnt2, xprev, wt, b, relu):
  n = xprev.shape[0]
  return pl.pallas_call(
      functools.partial(_combine_word_kernel, relu),
      grid=(n // BM,),
      in_specs=[pl.BlockSpec((BM, H), lambda i: (i, 0)),
                pl.BlockSpec((BM, 1), lambda i: (i, 0)),
                pl.BlockSpec((BM, H), lambda i: (i, 0)),
                pl.BlockSpec((BM, 1), lambda i: (i, 0)),
                pl.BlockSpec((BM, H), lambda i: (i, 0)),
                pl.BlockSpec((H, H), lambda i: (0, 0)),
                pl.BlockSpec((1, H), lambda i: (0, 0))],
      out_specs=pl.BlockSpec((BM, H), lambda i: (i, 0)),
      out_shape=jax.ShapeDtypeStruct((n, H), jnp.float32),
  )(acc1, cnt1, acc2, cnt2, xprev, wt, b)


def _combine_small_kernel(relu, a, c, xp, wlt, wrt, b, o_ref):
  mean = a[...] / jnp.maximum(c[...], 1.0)
  out = (jnp.dot(mean, wlt[...], preferred_element_type=jnp.float32)
         + jnp.dot(xp[...], wrt[...], preferred_element_type=jnp.float32)
         + b[...])
  o_ref[...] = jnp.maximum(out, 0.0) if relu else out


def _combine_small(acc, cnt, xprev, wlt, wrt, b, relu):
  n = xprev.shape[0]
  return pl.pallas_call(
      functools.partial(_combine_small_kernel, relu),
      grid=(n // BM,),
      in_specs=[pl.BlockSpec((BM, H), lambda i: (i, 0)),
                pl.BlockSpec((BM, 1), lambda i: (i, 0)),
                pl.BlockSpec((BM, H), lambda i: (i, 0)),
                pl.BlockSpec((H, H), lambda i: (0, 0)),
                pl.BlockSpec((H, H), lambda i: (0, 0)),
                pl.BlockSpec((1, H), lambda i: (0, 0))],
      out_specs=pl.BlockSpec((BM, H), lambda i: (i, 0)),
      out_shape=jax.ShapeDtypeStruct((n, H), jnp.float32),
  )(acc, cnt, xprev, wlt, wrt, b)


def _dot_kernel(a_ref, b_ref, o_ref):
  o_ref[...] = jnp.sum(a_ref[...] * b_ref[...], axis=1, keepdims=True)


def _rowdot(a, b):
  n = a.shape[0]
  out = pl.pallas_call(
      _dot_kernel,
      grid=(n // BM,),
      in_specs=[pl.BlockSpec((BM, H), lambda i: (i, 0)),
                pl.BlockSpec((BM, H), lambda i: (i, 0))],
      out_specs=pl.BlockSpec((BM, 1), lambda i: (i, 0)),
      out_shape=jax.ShapeDtypeStruct((n, 1), jnp.float32),
  )(a, b)
  return out.reshape(-1)


EPAD2 = 200320
EPAD1 = 100096
LPAD = 102400

_agg_ew = _make_agg(N_ENT, N_WORD, EPAD2)
_agg_sw = _make_agg(N_SENT, N_WORD, EPAD1)
_agg_we = _make_agg(N_WORD, N_ENT, EPAD2)
_agg_ws = _make_agg(N_WORD, N_SENT, EPAD1)
_pair_gather = _make_pair_gather(N_WORD, N_ENT, LPAD)


def kernel(word_x, word_node_id, entity_node_id, sentence_node_id,
           ei_word_entity, ei_entity_word, ei_word_sentence, ei_sentence_word,
           edge_label_index, params):
  p = params
  xe = p['entity_emb']
  xs = p['entity_emb'][:N_SENT]
  wx = jnp.pad(word_x, ((0, 0), (0, 9)))
  wt = jnp.pad(p['word_lin_W'].T, ((0, 9), (0, 0)))
  xw = _proj(wx, wt, p['word_lin_b'][None, :], p['word_emb'])

  def pad_edges(ei, n_pad_e):
    e = ei.shape[1]
    pad = n_pad_e - e
    spread = (jnp.arange(pad, dtype=jnp.int32) & 63)
    sentinel = jnp.full((pad,), 10 * DEV_CHUNK, jnp.int32)
    return (jnp.concatenate([ei[0], spread]),
            jnp.concatenate([ei[1], sentinel]))

  we_s, we_d = pad_edges(ei_word_entity, EPAD2)
  ew_s, ew_d = pad_edges(ei_entity_word, EPAD2)
  ws_s, ws_d = pad_edges(ei_word_sentence, EPAD1)
  sw_s, sw_d = pad_edges(ei_sentence_word, EPAD1)
  la = jnp.concatenate([edge_label_index[0],
                        (jnp.arange(LPAD - 100000, dtype=jnp.int32) & 63)])
  lb = jnp.concatenate([edge_label_index[1],
                        (jnp.arange(LPAD - 100000, dtype=jnp.int32) & 63)])

  def chained(fn, *args):
    nonlocal tok
    first = lax.optimization_barrier((args[0], tok))[0]
    out = fn(first, *args[1:])
    tok = out[0, 0]
    return out

  tok = jnp.float32(0.0)

  ones_tab = jnp.ones((N_WORD, H), jnp.float32) * (1.0 + 0.0 * word_x[0, 0])
  def _cnt_jax(dstx, n_dst):
    return jax.ops.segment_sum(jnp.ones((dstx.shape[0],), jnp.float32), dstx,
                               num_segments=n_dst)[:, None]

  cnt_ew = chained(_agg_ew, ones_tab[:N_ENT], ew_s, ew_d)[:N_WORD, 0:1]
  cnt_sw = chained(_agg_sw, ones_tab[:N_SENT], sw_s, sw_d)[:N_WORD, 0:1]
  cnt_we = chained(_agg_we, ones_tab, we_s, we_d)[:N_ENT, 0:1]
  cnt_ws = _cnt_jax(ei_word_sentence[1], N_SENT)

  hw_prev, he_prev, hs_prev = xw, xe, xs
  for lyr, relu in (('l1', True), ('l2', False)):
    def g(rel, s):
      return p[lyr + '_' + rel + '_' + s]

    y_ew = _matmul(he_prev, g('ew', 'Wl').T)
    y_sw = _matmul(hs_prev, g('sw', 'Wl').T)
    acc_ew = chained(_agg_ew, y_ew, ew_s, ew_d)
    acc_sw = chained(_agg_sw, y_sw, sw_s, sw_d)
    acc_we = chained(_agg_we, hw_prev, we_s, we_d)

    wr_sum = (g('ew', 'Wr') + g('sw', 'Wr')).T
    b_sum = (g('ew', 'bl') + g('sw', 'bl'))[None, :]
    hw = _combine_word(acc_ew[:N_WORD], cnt_ew,
                       acc_sw[:N_WORD], cnt_sw,
                       hw_prev, wr_sum, b_sum, relu)
    he = _combine_small(acc_we[:N_ENT], cnt_we, he_prev,
                        g('we', 'Wl').T, g('we', 'Wr').T,
                        g('we', 'bl')[None, :], relu)
    if relu:
      acc_ws = chained(_agg_ws, hw_prev, ws_s, ws_d)
      hs = _combine_small(acc_ws[:N_SENT], cnt_ws, hs_prev,
                          g('ws', 'Wl').T, g('ws', 'Wr').T,
                          g('ws', 'bl')[None, :], relu)
      hs_prev = hs
    hw_prev, he_prev = hw, he

  hw_prev = lax.optimization_barrier((hw_prev, tok))[0]
  ga, gb = _pair_gather(hw_prev, he_prev, la, lb)
  return _rowdot(ga[:100000], gb[:100000])
